# baseline (speedup 1.0000x reference)
import functools

import jax
import jax.numpy as jnp
from jax.experimental import pallas as pl
from jax.experimental.pallas import tpu as pltpu


def _round_up(x: int, m: int) -> int:
    return ((x + m - 1) // m) * m


def _disc_kernel(z_ref, rz_ref, w1_ref, b1_ref, w2_ref, b2_ref,
                 logits_ref, mid_ref, w1_s, w2_s, zz_s, *, negative_slope):
    @pl.when(pl.program_id(0) == 0)
    def _():
        w1_s[...] = w1_ref[...].astype(jnp.bfloat16)
        nc = w2_ref.shape[1]
        w2_s[...] = jnp.pad(w2_ref[...].astype(jnp.bfloat16),
                            ((0, 0), (0, w2_s.shape[1] - nc)))

    in_features = z_ref.shape[1]
    zz_s[:, :in_features] = z_ref[...].astype(jnp.bfloat16)
    zz_s[:, in_features:] = rz_ref[...].astype(jnp.bfloat16)

    out_pad = w1_s.shape[1]
    n_chunk = 256 if out_pad % 256 == 0 else out_pad
    acc = b2_ref[0, 0] * jnp.ones((z_ref.shape[0], w2_s.shape[1]),
                                  jnp.float32)
    for c in range(0, out_pad, n_chunk):
        cols = pl.ds(c, n_chunk)
        h = (jnp.dot(zz_s[...], w1_s[:, cols],
                     preferred_element_type=jnp.float32)
             + b1_ref[:, cols])
        mid = jnp.maximum(h, negative_slope * h)
        mid_ref[:, cols] = mid
        acc = acc + jnp.dot(mid.astype(jnp.bfloat16), w2_s[cols, :],
                            preferred_element_type=jnp.float32)
    logits_ref[...] = acc


def kernel(Z, rec_Z, w1, b1, w2, b2):
    B, in_features = Z.shape
    out_features = w1.shape[1]
    n_classes = w2.shape[1]

    OUT_PAD = _round_up(out_features, 128)
    if OUT_PAD != out_features:
        w1 = jnp.pad(w1, ((0, 0), (0, OUT_PAD - out_features)))
        b1 = jnp.pad(b1, ((0, 0), (0, OUT_PAD - out_features)))
        w2 = jnp.pad(w2, ((0, OUT_PAD - out_features), (0, 0)))
    NC_PAD = _round_up(n_classes, 128)

    VMEM_BUDGET = 100 * 1024 * 1024
    tile_b = min(1024, _round_up(B, 8))

    def _tile_bytes(tb):
        per_row = (2 * in_features + OUT_PAD + NC_PAD) * 4
        weights = (2 * in_features * OUT_PAD) * 5 \
            + OUT_PAD * NC_PAD * 2 + OUT_PAD * 8
        return 2 * tb * per_row + weights + tb * 2 * in_features * 2
    while tile_b > 8 and _tile_bytes(tile_b) > VMEM_BUDGET:
        tile_b //= 2
    tile_b = max(tile_b, 8)

    B_pad = _round_up(B, tile_b)
    if B_pad != B:
        Z_in = jnp.pad(Z, ((0, B_pad - B), (0, 0)))
        R_in = jnp.pad(rec_Z, ((0, B_pad - B), (0, 0)))
    else:
        Z_in, R_in = Z, rec_Z

    grid = (B_pad // tile_b,)

    body = functools.partial(_disc_kernel, negative_slope=0.2)

    flops = 2 * B_pad * (2 * in_features * OUT_PAD + OUT_PAD * NC_PAD)
    bytes_accessed = (
        4 * 2 * B_pad * in_features
        + 4 * (2 * in_features * OUT_PAD + OUT_PAD * n_classes)
        + 4 * (OUT_PAD + n_classes)
        + 4 * B_pad * (OUT_PAD + NC_PAD))

    logits_p, mid_p = pl.pallas_call(
        body,
        out_shape=(
            jax.ShapeDtypeStruct((B_pad, NC_PAD), jnp.float32),
            jax.ShapeDtypeStruct((B_pad, OUT_PAD), jnp.float32),
        ),
        grid=grid,
        in_specs=[
            pl.BlockSpec((tile_b, in_features), lambda i: (i, 0)),
            pl.BlockSpec((tile_b, in_features), lambda i: (i, 0)),
            pl.BlockSpec((2 * in_features, OUT_PAD), lambda i: (0, 0)),
            pl.BlockSpec((1, OUT_PAD), lambda i: (0, 0)),
            pl.BlockSpec((OUT_PAD, n_classes), lambda i: (0, 0)),
            pl.BlockSpec(memory_space=pltpu.SMEM),
        ],
        out_specs=(
            pl.BlockSpec((tile_b, NC_PAD), lambda i: (i, 0)),
            pl.BlockSpec((tile_b, OUT_PAD), lambda i: (i, 0)),
        ),
        scratch_shapes=[
            pltpu.VMEM((2 * in_features, OUT_PAD), jnp.bfloat16),
            pltpu.VMEM((OUT_PAD, NC_PAD), jnp.bfloat16),
            pltpu.VMEM((tile_b, 2 * in_features), jnp.bfloat16),
        ],
        compiler_params=pltpu.CompilerParams(
            dimension_semantics=("arbitrary",),
            vmem_limit_bytes=VMEM_BUDGET,
        ),
        cost_estimate=pl.CostEstimate(
            flops=flops, transcendentals=0, bytes_accessed=bytes_accessed),
    )(Z_in, R_in, w1, b1, w2, b2)

    return logits_p[:B, :n_classes], mid_p[:B, :out_features]


# speedup vs baseline: 1.2618x; 1.2618x over previous
import functools

import jax
import jax.numpy as jnp
from jax.experimental import pallas as pl
from jax.experimental.pallas import tpu as pltpu


def _round_up(x: int, m: int) -> int:
    return ((x + m - 1) // m) * m


def _disc_kernel(z_ref, rz_ref, w1_ref, b1_ref, w2_ref, b2_ref,
                 logits_ref, mid_ref, w1_s, w2_s, zz_s, *, negative_slope):
    @pl.when(pl.program_id(0) == 0)
    def _():
        w1_s[...] = w1_ref[...].astype(jnp.bfloat16)
        nc = w2_ref.shape[1]
        w2_s[...] = jnp.pad(w2_ref[...].astype(jnp.bfloat16),
                            ((0, 0), (0, w2_s.shape[1] - nc)))

    in_features = z_ref.shape[1]
    zz_s[:, :in_features] = z_ref[...].astype(jnp.bfloat16)
    zz_s[:, in_features:] = rz_ref[...].astype(jnp.bfloat16)

    h = (jnp.dot(zz_s[...], w1_s[...], preferred_element_type=jnp.float32)
         + b1_ref[...])

    mid = jnp.maximum(h, negative_slope * h)
    mid_ref[...] = mid

    logits = (jnp.dot(mid.astype(jnp.bfloat16), w2_s[...],
                      preferred_element_type=jnp.float32)
              + b2_ref[0, 0])
    logits_ref[...] = logits


def kernel(Z, rec_Z, w1, b1, w2, b2):
    B, in_features = Z.shape
    out_features = w1.shape[1]
    n_classes = w2.shape[1]

    OUT_PAD = _round_up(out_features, 128)
    if OUT_PAD != out_features:
        w1 = jnp.pad(w1, ((0, 0), (0, OUT_PAD - out_features)))
        b1 = jnp.pad(b1, ((0, 0), (0, OUT_PAD - out_features)))
        w2 = jnp.pad(w2, ((0, OUT_PAD - out_features), (0, 0)))
    NC_PAD = _round_up(n_classes, 128)

    VMEM_BUDGET = 100 * 1024 * 1024
    tile_b = min(512, _round_up(B, 8))

    def _tile_bytes(tb):
        per_row = (2 * in_features + OUT_PAD + NC_PAD) * 4
        weights = (2 * in_features * OUT_PAD) * 5 \
            + OUT_PAD * NC_PAD * 2 + OUT_PAD * 8
        return 2 * tb * per_row + weights + tb * 2 * in_features * 2
    while tile_b > 8 and _tile_bytes(tile_b) > VMEM_BUDGET:
        tile_b //= 2
    tile_b = max(tile_b, 8)

    B_pad = _round_up(B, tile_b)
    if B_pad != B:
        Z_in = jnp.pad(Z, ((0, B_pad - B), (0, 0)))
        R_in = jnp.pad(rec_Z, ((0, B_pad - B), (0, 0)))
    else:
        Z_in, R_in = Z, rec_Z

    grid = (B_pad // tile_b,)

    body = functools.partial(_disc_kernel, negative_slope=0.2)

    flops = 2 * B_pad * (2 * in_features * OUT_PAD + OUT_PAD * NC_PAD)
    bytes_accessed = (
        4 * 2 * B_pad * in_features
        + 4 * (2 * in_features * OUT_PAD + OUT_PAD * n_classes)
        + 4 * (OUT_PAD + n_classes)
        + 4 * B_pad * (OUT_PAD + NC_PAD))

    logits_p, mid_p = pl.pallas_call(
        body,
        out_shape=(
            jax.ShapeDtypeStruct((B_pad, NC_PAD), jnp.float32),
            jax.ShapeDtypeStruct((B_pad, OUT_PAD), jnp.float32),
        ),
        grid=grid,
        in_specs=[
            pl.BlockSpec((tile_b, in_features), lambda i: (i, 0)),
            pl.BlockSpec((tile_b, in_features), lambda i: (i, 0)),
            pl.BlockSpec((2 * in_features, OUT_PAD), lambda i: (0, 0)),
            pl.BlockSpec((1, OUT_PAD), lambda i: (0, 0)),
            pl.BlockSpec((OUT_PAD, n_classes), lambda i: (0, 0)),
            pl.BlockSpec(memory_space=pltpu.SMEM),
        ],
        out_specs=(
            pl.BlockSpec((tile_b, NC_PAD), lambda i: (i, 0)),
            pl.BlockSpec((tile_b, OUT_PAD), lambda i: (i, 0)),
        ),
        scratch_shapes=[
            pltpu.VMEM((2 * in_features, OUT_PAD), jnp.bfloat16),
            pltpu.VMEM((OUT_PAD, NC_PAD), jnp.bfloat16),
            pltpu.VMEM((tile_b, 2 * in_features), jnp.bfloat16),
        ],
        compiler_params=pltpu.CompilerParams(
            dimension_semantics=("arbitrary",),
            vmem_limit_bytes=VMEM_BUDGET,
        ),
        cost_estimate=pl.CostEstimate(
            flops=flops, transcendentals=0, bytes_accessed=bytes_accessed),
    )(Z_in, R_in, w1, b1, w2, b2)

    return logits_p[:B, :n_classes], mid_p[:B, :out_features]


# speedup vs baseline: 1.3214x; 1.0472x over previous
import functools

import jax
import jax.numpy as jnp
from jax.experimental import pallas as pl
from jax.experimental.pallas import tpu as pltpu


def _round_up(x: int, m: int) -> int:
    return ((x + m - 1) // m) * m


def _disc_kernel(z_ref, rz_ref, w1_ref, b1_ref, w2_ref, b2_ref,
                 logits_ref, mid_ref, w1_s, w2_s, zz_s, *, negative_slope):
    @pl.when(pl.program_id(0) == 0)
    def _():
        w1_s[...] = w1_ref[...].astype(jnp.bfloat16)
        nc = w2_ref.shape[1]
        w2_s[...] = jnp.pad(w2_ref[...].astype(jnp.bfloat16),
                            ((0, 0), (0, w2_s.shape[1] - nc)))

    in_features = z_ref.shape[1]
    zz_s[:, :in_features] = z_ref[...].astype(jnp.bfloat16)
    zz_s[:, in_features:] = rz_ref[...].astype(jnp.bfloat16)

    h = (jnp.dot(zz_s[...], w1_s[...], preferred_element_type=jnp.float32)
         + b1_ref[...])

    mid = jnp.maximum(h, negative_slope * h)
    mid_ref[...] = mid

    logits = (jnp.dot(mid.astype(jnp.bfloat16), w2_s[...],
                      preferred_element_type=jnp.float32)
              + b2_ref[0, 0])
    logits_ref[...] = logits


def kernel(Z, rec_Z, w1, b1, w2, b2):
    B, in_features = Z.shape
    out_features = w1.shape[1]
    n_classes = w2.shape[1]

    OUT_PAD = _round_up(out_features, 128)
    if OUT_PAD != out_features:
        w1 = jnp.pad(w1, ((0, 0), (0, OUT_PAD - out_features)))
        b1 = jnp.pad(b1, ((0, 0), (0, OUT_PAD - out_features)))
        w2 = jnp.pad(w2, ((0, OUT_PAD - out_features), (0, 0)))
    NC_PAD = _round_up(n_classes, 128)

    VMEM_BUDGET = 100 * 1024 * 1024
    tile_b = min(1024, _round_up(B, 8))

    def _tile_bytes(tb):
        per_row = (2 * in_features + OUT_PAD + NC_PAD) * 4
        weights = (2 * in_features * OUT_PAD) * 5 \
            + OUT_PAD * NC_PAD * 2 + OUT_PAD * 8
        return 2 * tb * per_row + weights + tb * 2 * in_features * 2
    while tile_b > 8 and _tile_bytes(tile_b) > VMEM_BUDGET:
        tile_b //= 2
    tile_b = max(tile_b, 8)

    B_pad = _round_up(B, tile_b)
    if B_pad != B:
        Z_in = jnp.pad(Z, ((0, B_pad - B), (0, 0)))
        R_in = jnp.pad(rec_Z, ((0, B_pad - B), (0, 0)))
    else:
        Z_in, R_in = Z, rec_Z

    grid = (B_pad // tile_b,)

    body = functools.partial(_disc_kernel, negative_slope=0.2)

    flops = 2 * B_pad * (2 * in_features * OUT_PAD + OUT_PAD * NC_PAD)
    bytes_accessed = (
        4 * 2 * B_pad * in_features
        + 4 * (2 * in_features * OUT_PAD + OUT_PAD * n_classes)
        + 4 * (OUT_PAD + n_classes)
        + 4 * B_pad * (OUT_PAD + NC_PAD))

    logits_p, mid_p = pl.pallas_call(
        body,
        out_shape=(
            jax.ShapeDtypeStruct((B_pad, NC_PAD), jnp.float32),
            jax.ShapeDtypeStruct((B_pad, OUT_PAD), jnp.float32),
        ),
        grid=grid,
        in_specs=[
            pl.BlockSpec((tile_b, in_features), lambda i: (i, 0)),
            pl.BlockSpec((tile_b, in_features), lambda i: (i, 0)),
            pl.BlockSpec((2 * in_features, OUT_PAD), lambda i: (0, 0)),
            pl.BlockSpec((1, OUT_PAD), lambda i: (0, 0)),
            pl.BlockSpec((OUT_PAD, n_classes), lambda i: (0, 0)),
            pl.BlockSpec(memory_space=pltpu.SMEM),
        ],
        out_specs=(
            pl.BlockSpec((tile_b, NC_PAD), lambda i: (i, 0)),
            pl.BlockSpec((tile_b, OUT_PAD), lambda i: (i, 0)),
        ),
        scratch_shapes=[
            pltpu.VMEM((2 * in_features, OUT_PAD), jnp.bfloat16),
            pltpu.VMEM((OUT_PAD, NC_PAD), jnp.bfloat16),
            pltpu.VMEM((tile_b, 2 * in_features), jnp.bfloat16),
        ],
        compiler_params=pltpu.CompilerParams(
            dimension_semantics=("arbitrary",),
            vmem_limit_bytes=VMEM_BUDGET,
        ),
        cost_estimate=pl.CostEstimate(
            flops=flops, transcendentals=0, bytes_accessed=bytes_accessed),
    )(Z_in, R_in, w1, b1, w2, b2)

    return logits_p[:B, :n_classes], mid_p[:B, :out_features]
